# baseline (speedup 1.0000x reference)
import functools

import jax
import jax.numpy as jnp
from jax import lax
from jax.experimental import pallas as pl
from jax.experimental.pallas import tpu as pltpu
from jax.experimental.pallas import tpu_sc as plsc

MAXLEN = 200
EMBED = 64
CH = 128
NBUF = 8
AHEAD = 4


def _make_gather(total_rows, vocab):
    info = plsc.get_sparse_core_info()
    nc, ns = info.num_cores, info.num_subcores
    nw = nc * ns
    rows_w = total_rows // nw
    nch = rows_w // CH
    assert total_rows % (nw * CH) == 0 and nch % NBUF == 0

    mesh = plsc.VectorSubcoreMesh(core_axis_name="c", subcore_axis_name="s")

    @functools.partial(
        pl.kernel,
        out_type=jax.ShapeDtypeStruct((total_rows, EMBED), jnp.float32),
        mesh=mesh,
        scratch_types=[
            pltpu.VMEM((nch, CH), jnp.int32),
            [pltpu.VMEM((CH, EMBED), jnp.float32) for _ in range(NBUF)],
            [pltpu.SemaphoreType.DMA for _ in range(NBUF)],
            [pltpu.SemaphoreType.DMA for _ in range(NBUF)],
        ],
        compiler_params=pltpu.CompilerParams(use_tc_tiling_on_sc=False),
    )
    def kern(xf_hbm, tok_hbm, g_hbm, idx_v, bufs, gsems, osems):
        wid = lax.axis_index("s") * nc + lax.axis_index("c")
        r0 = wid * rows_w

        pltpu.sync_copy(xf_hbm.at[pl.ds(wid * nch, nch)], idx_v)

        def gath(c, b):
            return pltpu.make_async_copy(
                tok_hbm.at[idx_v.at[c]], bufs[b], gsems[b])

        def put(c, b):
            return pltpu.make_async_copy(
                bufs[b], g_hbm.at[pl.ds(r0 + c * CH, CH)], osems[b])

        for c in range(AHEAD):
            gath(c, c % NBUF).start()

        def body(t0, _):
            for b in range(NBUF):
                t = t0 * NBUF + b
                gath(t, b).wait()
                put(t, b).start()
                nxt = t + AHEAD
                bn = (b + AHEAD) % NBUF

                @pl.when(nxt < nch)
                def _():
                    @pl.when(nxt >= NBUF)
                    def _():
                        put(nxt - NBUF, bn).wait()
                    gath(nxt, bn).start()
            return 0

        lax.fori_loop(0, nch // NBUF, body, 0, unroll=False)

        for b in range(NBUF):
            put(nch - NBUF + b, b).wait()

    return kern


def _addpos(gr, pos_table):
    maxlen, batch, embed = gr.shape

    def tck(g_ref, p_ref, o_ref):
        o_ref[0] = jnp.transpose(g_ref[0], (1, 0)) + p_ref[0, 0][:, None]

    return pl.pallas_call(
        tck,
        grid=(maxlen,),
        in_specs=[
            pl.BlockSpec((1, batch, embed), lambda j: (j, 0, 0)),
            pl.BlockSpec((1, 1, embed), lambda j: (j, 0, 0)),
        ],
        out_specs=pl.BlockSpec((1, embed, batch), lambda j: (j, 0, 0)),
        out_shape=jax.ShapeDtypeStruct((maxlen, embed, batch), jnp.float32),
    )(gr, pos_table.reshape(maxlen, 1, embed))


def kernel(x, token_table, pos_table):
    batch, maxlen = x.shape
    vocab, embed = token_table.shape
    assert maxlen == MAXLEN and embed == EMBED

    xt = jnp.transpose(x).astype(jnp.int32)
    xf = xt.reshape(maxlen * batch // CH, CH)
    g = _make_gather(maxlen * batch, vocab)(xf, token_table)
    gr = g.reshape(maxlen, batch, embed)
    ot = _addpos(gr, pos_table)
    return jnp.transpose(ot, (2, 0, 1))


# speedup vs baseline: 1.0007x; 1.0007x over previous
import functools

import jax
import jax.numpy as jnp
from jax import lax
from jax.experimental import pallas as pl
from jax.experimental.pallas import tpu as pltpu
from jax.experimental.pallas import tpu_sc as plsc

MAXLEN = 200
EMBED = 64
CH = 128
NBUF = 8
AHEAD = 7


def _make_gather(total_rows, vocab):
    info = plsc.get_sparse_core_info()
    nc, ns = info.num_cores, info.num_subcores
    nw = nc * ns
    rows_w = total_rows // nw
    nch = rows_w // CH
    assert total_rows % (nw * CH) == 0 and nch % NBUF == 0

    mesh = plsc.VectorSubcoreMesh(core_axis_name="c", subcore_axis_name="s")

    @functools.partial(
        pl.kernel,
        out_type=jax.ShapeDtypeStruct((total_rows, EMBED), jnp.float32),
        mesh=mesh,
        scratch_types=[
            pltpu.VMEM((nch, CH), jnp.int32),
            [pltpu.VMEM((CH, EMBED), jnp.float32) for _ in range(NBUF)],
            [pltpu.SemaphoreType.DMA for _ in range(NBUF)],
            [pltpu.SemaphoreType.DMA for _ in range(NBUF)],
        ],
        compiler_params=pltpu.CompilerParams(use_tc_tiling_on_sc=False),
    )
    def kern(xf_hbm, tok_hbm, g_hbm, idx_v, bufs, gsems, osems):
        wid = lax.axis_index("s") * nc + lax.axis_index("c")
        r0 = wid * rows_w

        pltpu.sync_copy(xf_hbm.at[pl.ds(wid * nch, nch)], idx_v)

        def gath(c, b):
            return pltpu.make_async_copy(
                tok_hbm.at[idx_v.at[c]], bufs[b], gsems[b])

        def put(c, b):
            return pltpu.make_async_copy(
                bufs[b], g_hbm.at[pl.ds(r0 + c * CH, CH)], osems[b])

        for c in range(AHEAD):
            gath(c, c % NBUF).start()

        def body(t0, _):
            for b in range(NBUF):
                t = t0 * NBUF + b
                gath(t, b).wait()
                put(t, b).start()
                nxt = t + AHEAD
                bn = (b + AHEAD) % NBUF

                @pl.when(nxt < nch)
                def _():
                    @pl.when(nxt >= NBUF)
                    def _():
                        put(nxt - NBUF, bn).wait()
                    gath(nxt, bn).start()
            return 0

        lax.fori_loop(0, nch // NBUF, body, 0, unroll=False)

        for b in range(NBUF):
            put(nch - NBUF + b, b).wait()

    return kern


def _addpos(gr, pos_table):
    maxlen, batch, embed = gr.shape

    def tck(g_ref, p_ref, o_ref):
        o_ref[0] = jnp.transpose(g_ref[0], (1, 0)) + p_ref[0, 0][:, None]

    return pl.pallas_call(
        tck,
        grid=(maxlen,),
        in_specs=[
            pl.BlockSpec((1, batch, embed), lambda j: (j, 0, 0)),
            pl.BlockSpec((1, 1, embed), lambda j: (j, 0, 0)),
        ],
        out_specs=pl.BlockSpec((1, embed, batch), lambda j: (j, 0, 0)),
        out_shape=jax.ShapeDtypeStruct((maxlen, embed, batch), jnp.float32),
    )(gr, pos_table.reshape(maxlen, 1, embed))


def kernel(x, token_table, pos_table):
    batch, maxlen = x.shape
    vocab, embed = token_table.shape
    assert maxlen == MAXLEN and embed == EMBED

    xt = jnp.transpose(x).astype(jnp.int32)
    xf = xt.reshape(maxlen * batch // CH, CH)
    g = _make_gather(maxlen * batch, vocab)(xf, token_table)
    gr = g.reshape(maxlen, batch, embed)
    ot = _addpos(gr, pos_table)
    return jnp.transpose(ot, (2, 0, 1))


# speedup vs baseline: 1.0549x; 1.0542x over previous
import functools

import jax
import jax.numpy as jnp
from jax import lax
from jax.experimental import pallas as pl
from jax.experimental.pallas import tpu as pltpu
from jax.experimental.pallas import tpu_sc as plsc

MAXLEN = 200
EMBED = 64
CH = 128
K = 1
NBUF = 8
AHEAD = 7


def _make_gather(total_rows, vocab):
    info = plsc.get_sparse_core_info()
    nc, ns = info.num_cores, info.num_subcores
    nw = nc * ns
    rows_w = total_rows // nw
    nch = rows_w // CH
    nsc = nch // K
    assert total_rows % (nw * CH * K) == 0 and nsc % NBUF == 0

    mesh = plsc.VectorSubcoreMesh(core_axis_name="c", subcore_axis_name="s")

    @functools.partial(
        pl.kernel,
        out_type=jax.ShapeDtypeStruct((total_rows // CH, CH, EMBED),
                                      jnp.float32),
        mesh=mesh,
        scratch_types=[
            pltpu.VMEM((nch, CH), jnp.int32),
            [pltpu.VMEM((CH, EMBED), jnp.float32) for _ in range(NBUF)],
            [pltpu.SemaphoreType.DMA for _ in range(NBUF)],
            [pltpu.SemaphoreType.DMA for _ in range(NBUF)],
        ],
        compiler_params=pltpu.CompilerParams(use_tc_tiling_on_sc=False),
    )
    def kern(xf_hbm, tok_hbm, g_hbm, idx_v, bufs, gsems, osems):
        wid = lax.axis_index("s") * nc + lax.axis_index("c")
        c0 = wid * nch

        pltpu.sync_copy(xf_hbm.at[pl.ds(c0, nch)], idx_v)

        def gath(c, b):
            return pltpu.make_async_copy(
                tok_hbm.at[idx_v.at[c]], bufs[b], gsems[b])

        def put(c, b):
            return pltpu.make_async_copy(
                bufs[b], g_hbm.at[c0 + c], osems[b])

        for c in range(AHEAD):
            gath(c, c % NBUF).start()

        def body(t0, _):
            for b in range(NBUF):
                t = t0 * NBUF + b
                gath(t, b).wait()
                nxt = t + AHEAD
                bn = (b + AHEAD) % NBUF

                @pl.when(nxt < nsc)
                def _():
                    gath(nxt, bn).start()
            return 0

        lax.fori_loop(0, nsc // NBUF, body, 0, unroll=False)

        put(nsc - 1, (nsc - 1) % NBUF).start()
        put(nsc - 1, (nsc - 1) % NBUF).wait()

    return kern


def _addpos(gr, pos_table):
    maxlen, batch, embed = gr.shape

    def tck(g_ref, p_ref, o_ref):
        o_ref[0] = jnp.transpose(g_ref[0], (1, 0)) + p_ref[0, 0][:, None]

    return pl.pallas_call(
        tck,
        grid=(maxlen,),
        in_specs=[
            pl.BlockSpec((1, batch, embed), lambda j: (j, 0, 0)),
            pl.BlockSpec((1, 1, embed), lambda j: (j, 0, 0)),
        ],
        out_specs=pl.BlockSpec((1, embed, batch), lambda j: (j, 0, 0)),
        out_shape=jax.ShapeDtypeStruct((maxlen, embed, batch), jnp.float32),
    )(gr, pos_table.reshape(maxlen, 1, embed))


def kernel(x, token_table, pos_table):
    batch, maxlen = x.shape
    vocab, embed = token_table.shape
    assert maxlen == MAXLEN and embed == EMBED

    xt = jnp.transpose(x).astype(jnp.int32)
    xf = xt.reshape(maxlen * batch // CH, CH)
    g = _make_gather(maxlen * batch, vocab)(xf, token_table)
    gr = g.reshape(maxlen, batch, embed)
    ot = _addpos(gr, pos_table)
    return jnp.transpose(ot, (2, 0, 1))


# speedup vs baseline: 1.0684x; 1.0128x over previous
import functools

import jax
import jax.numpy as jnp
from jax import lax
from jax.experimental import pallas as pl
from jax.experimental.pallas import tpu as pltpu
from jax.experimental.pallas import tpu_sc as plsc

MAXLEN = 200
EMBED = 64
HALF = MAXLEN // 2
NBUF = 2


def _make_kernel(batch, vocab):
    info = plsc.get_sparse_core_info()
    nc, ns = info.num_cores, info.num_subcores
    nw = nc * ns
    rows_w = batch // nw
    assert batch % nw == 0

    mesh = plsc.VectorSubcoreMesh(core_axis_name="c", subcore_axis_name="s")

    @functools.partial(
        pl.kernel,
        out_type=jax.ShapeDtypeStruct((batch, MAXLEN, EMBED), jnp.float32),
        mesh=mesh,
        scratch_types=[
            pltpu.VMEM((rows_w, 2, HALF), jnp.int32),
            pltpu.VMEM((MAXLEN, EMBED), jnp.float32),
            [pltpu.VMEM((MAXLEN, EMBED), jnp.float32) for _ in range(NBUF)],
            [pltpu.VMEM((MAXLEN, EMBED), jnp.float32) for _ in range(NBUF)],
            [pltpu.SemaphoreType.DMA for _ in range(NBUF)],
            [pltpu.SemaphoreType.DMA for _ in range(NBUF)],
        ],
        compiler_params=pltpu.CompilerParams(use_tc_tiling_on_sc=False),
    )
    def kern(x_hbm, tok_hbm, pos_hbm, out_hbm,
             idx_v, pos_v, tokbufs, outbufs, gsems, osems):
        wid = lax.axis_index("s") * nc + lax.axis_index("c")
        r0 = wid * rows_w

        pltpu.sync_copy(x_hbm.at[pl.ds(r0, rows_w)], idx_v)
        pltpu.sync_copy(pos_hbm, pos_v)

        def gathers(r, b):
            return [
                pltpu.make_async_copy(
                    tok_hbm.at[idx_v.at[r, h]],
                    tokbufs[b].at[pl.ds(h * HALF, HALF)],
                    gsems[b])
                for h in range(2)
            ]

        def put(r, b):
            return pltpu.make_async_copy(
                outbufs[b], out_hbm.at[r0 + r], osems[b])

        for b in range(NBUF):
            for c in gathers(b, b):
                c.start()

        def body(ro, _):
            for b in range(NBUF):
                r = ro + b
                @pl.when(r >= NBUF)
                def _():
                    put(r - NBUF, b).wait()
                for c in gathers(r, b):
                    c.wait()

                def add_row(j, _):
                    for k in range(EMBED // 16):
                        sl = pl.ds(k * 16, 16)
                        outbufs[b][j, sl] = tokbufs[b][j, sl] + pos_v[j, sl]
                    return 0

                lax.fori_loop(0, MAXLEN, add_row, 0)

                @pl.when(r + NBUF < rows_w)
                def _():
                    for c in gathers(r + NBUF, b):
                        c.start()
                put(r, b).start()
            return 0

        lax.fori_loop(0, rows_w // NBUF, lambda i, c: body(i * NBUF, c), 0,
                      unroll=False)

        for b in range(NBUF):
            put(rows_w - NBUF + b, b).wait()

    return kern


def kernel(x, token_table, pos_table):
    batch, maxlen = x.shape
    vocab, embed = token_table.shape
    assert maxlen == MAXLEN and embed == EMBED
    xf = x.reshape(batch, 2, HALF).astype(jnp.int32)
    return _make_kernel(batch, vocab)(xf, token_table, pos_table)
